# baseline (speedup 1.0000x reference)
import jax
import jax.numpy as jnp
from jax import lax
from jax.experimental import pallas as pl

_NB, _NA, _NC, _NH, _NW = 16, 5, 80, 19, 19
_S = _NH * _NW
_NAS = _NA * _S
_NT = 50
_CH = 5 + _NC
_ANCHORS = [0.57273, 0.677385, 1.87446, 2.06253, 3.33843, 5.47434,
            7.88282, 3.52778, 9.77052, 9.16828]
_THRESH = 0.6
_OBJ_SCALE = 5.0


def _iou(ax, ay, aw, ah, bx, by, bw, bh):
    mx = jnp.minimum(ax - aw / 2.0, bx - bw / 2.0)
    Mx = jnp.maximum(ax + aw / 2.0, bx + bw / 2.0)
    my = jnp.minimum(ay - ah / 2.0, by - bh / 2.0)
    My = jnp.maximum(ay + ah / 2.0, by + bh / 2.0)
    uw = Mx - mx
    uh = My - my
    cw = aw + bw - uw
    ch = ah + bh - uh
    carea = jnp.where((cw <= 0) | (ch <= 0), 0.0, cw * ch)
    uarea = aw * ah + bw * bh - carea
    return carea / uarea


def _iou_gt_thresh(ax, ay, aw, ah, bx, by, bw, bh, thresh):
    mx = jnp.minimum(ax - aw / 2.0, bx - bw / 2.0)
    Mx = jnp.maximum(ax + aw / 2.0, bx + bw / 2.0)
    my = jnp.minimum(ay - ah / 2.0, by - bh / 2.0)
    My = jnp.maximum(ay + ah / 2.0, by + bh / 2.0)
    uw = Mx - mx
    uh = My - my
    cw = aw + bw - uw
    ch = ah + bh - uh
    carea = jnp.where((cw <= 0) | (ch <= 0), 0.0, cw * ch)
    uarea = aw * ah + bw * bh - carea
    return carea > thresh * uarea


def _dot(a, b, ca, cb):
    return lax.dot_general(a, b, dimension_numbers=(((ca,), (cb,)), ((), ())),
                           preferred_element_type=jnp.float32)


_BPG = 2


def _one_batch(dref, tref, i):
    f32 = jnp.float32
    a_row = lax.broadcasted_iota(jnp.int32, (1, _NA), 1)
    aw_r = jnp.zeros((1, _NA), f32)
    ah_r = jnp.zeros((1, _NA), f32)
    for a in range(_NA):
        aw_r = jnp.where(a_row == a, _ANCHORS[2 * a], aw_r)
        ah_r = jnp.where(a_row == a, _ANCHORS[2 * a + 1], ah_r)

    tb = tref[i]
    tcls = tb[:, 0:1]
    gx = tb[:, 1:2] * _NW
    gy = tb[:, 2:3] * _NH
    gw = tb[:, 3:4] * _NW
    gh = tb[:, 4:5] * _NH

    bad = (tb[:, 1:2] == 0).astype(f32)
    r_i = lax.broadcasted_iota(jnp.int32, (_NT, _NT), 0)
    c_i = lax.broadcasted_iota(jnp.int32, (_NT, _NT), 1)
    lower = (c_i <= r_i).astype(f32)
    pref_bad = _dot(lower, bad, 1, 0)
    validf = (pref_bad == 0).astype(f32)

    anc_iou = _iou(0.0, 0.0, aw_r, ah_r, 0.0, 0.0, gw, gh)
    amax = jnp.max(anc_iou, axis=1, keepdims=True)
    a_io = lax.broadcasted_iota(jnp.int32, (_NT, _NA), 1)
    bn = jnp.min(jnp.where(anc_iou == amax, a_io, _NA), axis=1, keepdims=True)
    an_oh = (a_io == bn).astype(f32)
    aw_sel = jnp.sum(an_oh * aw_r, axis=1, keepdims=True)
    ah_sel = jnp.sum(an_oh * ah_r, axis=1, keepdims=True)

    gi = jnp.floor(gx)
    gj = jnp.floor(gy)
    tx_val = gx - gi
    ty_val = gy - gj
    tw_val = jnp.where(validf > 0, jnp.log(gw / aw_sel), 0.0)
    th_val = jnp.where(validf > 0, jnp.log(gh / ah_sel), 0.0)

    cell_i = gj.astype(jnp.int32) * _NW + gi.astype(jnp.int32)
    cell_full = bn * _S + cell_i

    cf = cell_full.astype(f32)
    ones_c = jnp.ones((_NT, 1), f32)
    cell_row = _dot(ones_c, cf, 1, 1)
    valid_row = _dot(ones_c, validf, 1, 1)
    dup = jnp.sum(jnp.where((cell_row == cf) & (valid_row > 0)
                            & (c_i > r_i), 1.0, 0.0), axis=1, keepdims=True)
    winf = validf * (dup == 0).astype(f32)

    as_io = lax.broadcasted_iota(jnp.int32, (_NT, _NAS), 1)
    oh = (as_io == cell_full).astype(f32)
    s_io = lax.broadcasted_iota(jnp.int32, (_NT, _S), 1)
    oh_s = (s_io == cell_i).astype(f32)
    a_io5 = lax.broadcasted_iota(jnp.int32, (_NT, _NA), 1)
    an_oh = (a_io5 == bn).astype(f32)

    s_col = lax.broadcasted_iota(jnp.int32, (1, _NAS), 1)
    sp = s_col % _S
    fi = (sp % _NW).astype(f32)
    fj = (sp // _NW).astype(f32)
    awc = jnp.zeros((1, _NAS), f32)
    ahc = jnp.zeros((1, _NAS), f32)
    for a in range(_NA):
        sel = (s_col // _S) == a
        awc = jnp.where(sel, _ANCHORS[2 * a], awc)
        ahc = jnp.where(sel, _ANCHORS[2 * a + 1], ahc)

    def cat(c):
        return jnp.concatenate(
            [dref[i, _CH * a + c:_CH * a + c + 1, :] for a in range(_NA)],
            axis=1)
    xr = cat(0)
    yr = cat(1)
    wr = cat(2)
    hr = cat(3)
    cr = cat(4)
    x = 1.0 / (1.0 + jnp.exp(-xr))
    y = 1.0 / (1.0 + jnp.exp(-yr))
    conf = 1.0 / (1.0 + jnp.exp(-cr))
    px = x + fi
    py = y + fj
    pw = jnp.exp(wr) * awc
    ph = jnp.exp(hr) * ahc

    P = jnp.concatenate([px, py, pw, ph], axis=0)
    G = _dot(oh, P, 1, 1)
    pxc = G[:, 0:1]
    pyc = G[:, 1:2]
    pwc = G[:, 2:3]
    phc = G[:, 3:4]
    iou_val = _iou(gx, gy, gw, gh, pxc, pyc, pwc, phc)
    iou_val = jnp.where(validf > 0, iou_val, 0.0)

    onesf = jnp.ones((_NT, 1), f32)
    V = jnp.concatenate(
        [onesf, tx_val, ty_val, tw_val, th_val, iou_val], axis=1)

    gxz = gx * validf
    gyz = gy * validf
    gwz = gw * validf
    ghz = gh * validf
    hit = _iou_gt_thresh(px, py, pw, ph, gxz, gyz, gwz, ghz, _THRESH)
    noobj = jnp.where(jnp.max(hit.astype(f32), axis=0, keepdims=True) > 0,
                      0.0, 1.0)

    D = _dot(V * winf, oh, 0, 0)
    obj = D[0:1]
    txd = D[1:2] + 0.5 * (1.0 - obj)
    tyd = D[2:3] + 0.5 * (1.0 - obj)
    twd = D[3:4]
    thd = D[4:5]
    tcf = D[5:6]
    cmask = jnp.where(obj > 0, _OBJ_SCALE, noobj)

    lx = jnp.sum((x - txd) ** 2)
    ly = jnp.sum((y - tyd) ** 2)
    lw = jnp.sum((wr - twd) ** 2)
    lh = jnp.sum((hr - thd) ** 2)
    lconf = jnp.sum(cmask * (conf - tcf) ** 2)
    total = 0.5 * (lx + ly + lw + lh + lconf)

    L = jnp.zeros((_NT, _NC), f32)
    for a in range(_NA):
        cls_a = dref[i, _CH * a + 5:_CH * a + _CH, :]
        L = L + an_oh[:, a:a + 1] * _dot(oh_s, cls_a, 1, 1)
    mxL = jnp.max(L, axis=1, keepdims=True)
    seL = jnp.sum(jnp.exp(L - mxL), axis=1, keepdims=True)
    lseL = mxL + jnp.log(seL)
    c50 = lax.broadcasted_iota(jnp.int32, (_NT, _NC), 1)
    pickv = jnp.sum(jnp.where(c50 == tcls.astype(jnp.int32), L, 0.0),
                    axis=1, keepdims=True)
    return total + jnp.sum(winf * (lseL - pickv))


def _body(dref, tref, acc):
    b = pl.program_id(0)
    total = jnp.zeros((), jnp.float32)
    for i in range(_BPG):
        total = total + _one_batch(dref, tref, i)

    @pl.when(b == 0)
    def _init():
        acc[...] = jnp.zeros((1, 1), jnp.float32)

    acc[...] += total.reshape(1, 1)


def kernel(output, target):
    out3 = output.reshape(_NB, _NA * _CH, _S)
    tgt3 = target.reshape(_NB, _NT, 5)
    res = pl.pallas_call(
        _body,
        grid=(_NB // _BPG,),
        in_specs=[
            pl.BlockSpec((_BPG, _NA * _CH, _S), lambda b: (b, 0, 0)),
            pl.BlockSpec((_BPG, _NT, 5), lambda b: (b, 0, 0)),
        ],
        out_specs=pl.BlockSpec((1, 1), lambda b: (0, 0)),
        out_shape=jax.ShapeDtypeStruct((1, 1), jnp.float32),
    )(out3, tgt3)
    return res[0, 0]


# speedup vs baseline: 1.0495x; 1.0495x over previous
import jax
import jax.numpy as jnp
from jax import lax
from jax.experimental import pallas as pl

_NB, _NA, _NC, _NH, _NW = 16, 5, 80, 19, 19
_S = _NH * _NW
_NAS = _NA * _S
_NT = 50
_CH = 5 + _NC
_ANCHORS = [0.57273, 0.677385, 1.87446, 2.06253, 3.33843, 5.47434,
            7.88282, 3.52778, 9.77052, 9.16828]
_THRESH = 0.6
_OBJ_SCALE = 5.0


def _iou(ax, ay, aw, ah, bx, by, bw, bh):
    mx = jnp.minimum(ax - aw / 2.0, bx - bw / 2.0)
    Mx = jnp.maximum(ax + aw / 2.0, bx + bw / 2.0)
    my = jnp.minimum(ay - ah / 2.0, by - bh / 2.0)
    My = jnp.maximum(ay + ah / 2.0, by + bh / 2.0)
    uw = Mx - mx
    uh = My - my
    cw = aw + bw - uw
    ch = ah + bh - uh
    carea = jnp.where((cw <= 0) | (ch <= 0), 0.0, cw * ch)
    uarea = aw * ah + bw * bh - carea
    return carea / uarea


def _iou_gt_thresh(ax, ay, aw, ah, bx, by, bw, bh, thresh):
    mx = jnp.minimum(ax - aw / 2.0, bx - bw / 2.0)
    Mx = jnp.maximum(ax + aw / 2.0, bx + bw / 2.0)
    my = jnp.minimum(ay - ah / 2.0, by - bh / 2.0)
    My = jnp.maximum(ay + ah / 2.0, by + bh / 2.0)
    uw = Mx - mx
    uh = My - my
    cw = aw + bw - uw
    ch = ah + bh - uh
    carea = jnp.where((cw <= 0) | (ch <= 0), 0.0, cw * ch)
    uarea = aw * ah + bw * bh - carea
    return carea > thresh * uarea


def _dot(a, b, ca, cb):
    return lax.dot_general(a, b, dimension_numbers=(((ca,), (cb,)), ((), ())),
                           preferred_element_type=jnp.float32)


def _body(dref, tref, acc):
    b = pl.program_id(0)
    f32 = jnp.float32
    a_row = lax.broadcasted_iota(jnp.int32, (1, _NA), 1)
    aw_r = jnp.zeros((1, _NA), f32)
    ah_r = jnp.zeros((1, _NA), f32)
    for a in range(_NA):
        aw_r = jnp.where(a_row == a, _ANCHORS[2 * a], aw_r)
        ah_r = jnp.where(a_row == a, _ANCHORS[2 * a + 1], ah_r)

    tb = tref[0]
    tcls = tb[:, 0:1]
    gx = tb[:, 1:2] * _NW
    gy = tb[:, 2:3] * _NH
    gw = tb[:, 3:4] * _NW
    gh = tb[:, 4:5] * _NH

    bad = (tb[:, 1:2] == 0).astype(f32)
    r_i = lax.broadcasted_iota(jnp.int32, (_NT, _NT), 0)
    c_i = lax.broadcasted_iota(jnp.int32, (_NT, _NT), 1)
    lower = (c_i <= r_i).astype(f32)
    pref_bad = _dot(lower, bad, 1, 0)
    validf = (pref_bad == 0).astype(f32)

    anc_iou = _iou(0.0, 0.0, aw_r, ah_r, 0.0, 0.0, gw, gh)
    amax = jnp.max(anc_iou, axis=1, keepdims=True)
    a_io = lax.broadcasted_iota(jnp.int32, (_NT, _NA), 1)
    bn = jnp.min(jnp.where(anc_iou == amax, a_io, _NA), axis=1, keepdims=True)
    an_oh = (a_io == bn).astype(f32)
    aw_sel = jnp.sum(an_oh * aw_r, axis=1, keepdims=True)
    ah_sel = jnp.sum(an_oh * ah_r, axis=1, keepdims=True)

    gi = jnp.floor(gx)
    gj = jnp.floor(gy)
    tx_val = gx - gi
    ty_val = gy - gj
    tw_val = jnp.where(validf > 0, jnp.log(gw / aw_sel), 0.0)
    th_val = jnp.where(validf > 0, jnp.log(gh / ah_sel), 0.0)

    cell_i = gj.astype(jnp.int32) * _NW + gi.astype(jnp.int32)
    cell_full = bn * _S + cell_i

    cf = cell_full.astype(f32)
    ones_c = jnp.ones((_NT, 1), f32)
    cell_row = _dot(ones_c, cf, 1, 1)
    valid_row = _dot(ones_c, validf, 1, 1)
    dup = jnp.sum(jnp.where((cell_row == cf) & (valid_row > 0)
                            & (c_i > r_i), 1.0, 0.0), axis=1, keepdims=True)
    winf = validf * (dup == 0).astype(f32)

    as_io = lax.broadcasted_iota(jnp.int32, (_NT, _NAS), 1)
    oh = (as_io == cell_full).astype(f32)

    s_col = lax.broadcasted_iota(jnp.int32, (1, _NAS), 1)
    sp = s_col % _S
    fi = (sp % _NW).astype(f32)
    fj = (sp // _NW).astype(f32)
    awc = jnp.zeros((1, _NAS), f32)
    ahc = jnp.zeros((1, _NAS), f32)
    for a in range(_NA):
        sel = (s_col // _S) == a
        awc = jnp.where(sel, _ANCHORS[2 * a], awc)
        ahc = jnp.where(sel, _ANCHORS[2 * a + 1], ahc)

    def cat(c):
        return jnp.concatenate(
            [dref[0, _CH * a + c:_CH * a + c + 1, :] for a in range(_NA)],
            axis=1)
    xr = cat(0)
    yr = cat(1)
    wr = cat(2)
    hr = cat(3)
    cr = cat(4)
    x = 1.0 / (1.0 + jnp.exp(-xr))
    y = 1.0 / (1.0 + jnp.exp(-yr))
    conf = 1.0 / (1.0 + jnp.exp(-cr))
    px = x + fi
    py = y + fj
    pw = jnp.exp(wr) * awc
    ph = jnp.exp(hr) * ahc

    pxc = _dot(oh, px, 1, 1)
    pyc = _dot(oh, py, 1, 1)
    pwc = _dot(oh, pw, 1, 1)
    phc = _dot(oh, ph, 1, 1)
    iou_val = _iou(gx, gy, gw, gh, pxc, pyc, pwc, phc)
    iou_val = jnp.where(validf > 0, iou_val, 0.0)

    onesf = jnp.ones((_NT, 1), f32)
    V = jnp.concatenate(
        [onesf, tx_val, ty_val, tw_val, th_val, iou_val], axis=1)

    gxz = gx * validf
    gyz = gy * validf
    gwz = gw * validf
    ghz = gh * validf
    hit = _iou_gt_thresh(px, py, pw, ph, gxz, gyz, gwz, ghz, _THRESH)
    noobj = jnp.where(jnp.max(hit.astype(f32), axis=0, keepdims=True) > 0,
                      0.0, 1.0)

    D = _dot(V * winf, oh, 0, 0)
    obj = D[0:1]
    txd = D[1:2] + 0.5 * (1.0 - obj)
    tyd = D[2:3] + 0.5 * (1.0 - obj)
    twd = D[3:4]
    thd = D[4:5]
    tcf = D[5:6]
    cmask = jnp.where(obj > 0, _OBJ_SCALE, noobj)

    lx = jnp.sum((x - txd) ** 2)
    ly = jnp.sum((y - tyd) ** 2)
    lw = jnp.sum((wr - twd) ** 2)
    lh = jnp.sum((hr - thd) ** 2)
    lconf = jnp.sum(cmask * (conf - tcf) ** 2)
    total = 0.5 * (lx + ly + lw + lh + lconf)

    cls_all = jnp.concatenate(
        [dref[0, _CH * a + 5:_CH * a + _CH, :] for a in range(_NA)],
        axis=1)
    L = _dot(oh, cls_all, 1, 1)
    mxL = jnp.max(L, axis=1, keepdims=True)
    seL = jnp.sum(jnp.exp(L - mxL), axis=1, keepdims=True)
    lseL = mxL + jnp.log(seL)
    c50 = lax.broadcasted_iota(jnp.int32, (_NT, _NC), 1)
    pickv = jnp.sum(jnp.where(c50 == tcls.astype(jnp.int32), L, 0.0),
                    axis=1, keepdims=True)
    total = total + jnp.sum(winf * (lseL - pickv))

    @pl.when(b == 0)
    def _init():
        acc[...] = jnp.zeros((1, 1), f32)

    acc[...] += total.reshape(1, 1)


def kernel(output, target):
    out3 = output.reshape(_NB, _NA * _CH, _S)
    tgt3 = target.reshape(_NB, _NT, 5)
    res = pl.pallas_call(
        _body,
        grid=(_NB,),
        in_specs=[
            pl.BlockSpec((1, _NA * _CH, _S), lambda b: (b, 0, 0)),
            pl.BlockSpec((1, _NT, 5), lambda b: (b, 0, 0)),
        ],
        out_specs=pl.BlockSpec((1, 1), lambda b: (0, 0)),
        out_shape=jax.ShapeDtypeStruct((1, 1), jnp.float32),
    )(out3, tgt3)
    return res[0, 0]


# speedup vs baseline: 1.1045x; 1.0524x over previous
import jax
import jax.numpy as jnp
from jax import lax
from jax.experimental import pallas as pl

_NB, _NA, _NC, _NH, _NW = 16, 5, 80, 19, 19
_S = _NH * _NW
_NAS = _NA * _S
_NT = 50
_CH = 5 + _NC
_ANCHORS = [0.57273, 0.677385, 1.87446, 2.06253, 3.33843, 5.47434,
            7.88282, 3.52778, 9.77052, 9.16828]
_THRESH = 0.6
_OBJ_SCALE = 5.0


def _iou(ax, ay, aw, ah, bx, by, bw, bh):
    mx = jnp.minimum(ax - aw / 2.0, bx - bw / 2.0)
    Mx = jnp.maximum(ax + aw / 2.0, bx + bw / 2.0)
    my = jnp.minimum(ay - ah / 2.0, by - bh / 2.0)
    My = jnp.maximum(ay + ah / 2.0, by + bh / 2.0)
    uw = Mx - mx
    uh = My - my
    cw = aw + bw - uw
    ch = ah + bh - uh
    carea = jnp.where((cw <= 0) | (ch <= 0), 0.0, cw * ch)
    uarea = aw * ah + bw * bh - carea
    return carea / uarea


def _iou_gt_thresh(ax, ay, aw, ah, bx, by, bw, bh, thresh):
    mx = jnp.minimum(ax - aw / 2.0, bx - bw / 2.0)
    Mx = jnp.maximum(ax + aw / 2.0, bx + bw / 2.0)
    my = jnp.minimum(ay - ah / 2.0, by - bh / 2.0)
    My = jnp.maximum(ay + ah / 2.0, by + bh / 2.0)
    uw = Mx - mx
    uh = My - my
    cw = aw + bw - uw
    ch = ah + bh - uh
    carea = jnp.where((cw <= 0) | (ch <= 0), 0.0, cw * ch)
    uarea = aw * ah + bw * bh - carea
    return carea > thresh * uarea


def _dot(a, b, ca, cb):
    return lax.dot_general(a, b, dimension_numbers=(((ca,), (cb,)), ((), ())),
                           preferred_element_type=jnp.float32)


def _body(dref, tref, acc):
    b = pl.program_id(0)
    f32 = jnp.float32
    a_row = lax.broadcasted_iota(jnp.int32, (1, _NA), 1)
    aw_r = jnp.zeros((1, _NA), f32)
    ah_r = jnp.zeros((1, _NA), f32)
    for a in range(_NA):
        aw_r = jnp.where(a_row == a, _ANCHORS[2 * a], aw_r)
        ah_r = jnp.where(a_row == a, _ANCHORS[2 * a + 1], ah_r)

    tb = tref[0]
    tcls = tb[:, 0:1]
    gx = tb[:, 1:2] * _NW
    gy = tb[:, 2:3] * _NH
    gw = tb[:, 3:4] * _NW
    gh = tb[:, 4:5] * _NH

    bad = (tb[:, 1:2] == 0).astype(f32)
    r_i = lax.broadcasted_iota(jnp.int32, (_NT, _NT), 0)
    c_i = lax.broadcasted_iota(jnp.int32, (_NT, _NT), 1)
    lower = (c_i <= r_i).astype(f32)
    pref_bad = _dot(lower, bad, 1, 0)
    validf = (pref_bad == 0).astype(f32)

    anc_iou = _iou(0.0, 0.0, aw_r, ah_r, 0.0, 0.0, gw, gh)
    amax = jnp.max(anc_iou, axis=1, keepdims=True)
    a_io = lax.broadcasted_iota(jnp.int32, (_NT, _NA), 1)
    bn = jnp.min(jnp.where(anc_iou == amax, a_io, _NA), axis=1, keepdims=True)
    an_oh = (a_io == bn).astype(f32)
    aw_sel = jnp.sum(an_oh * aw_r, axis=1, keepdims=True)
    ah_sel = jnp.sum(an_oh * ah_r, axis=1, keepdims=True)

    gi = jnp.floor(gx)
    gj = jnp.floor(gy)
    tx_val = gx - gi
    ty_val = gy - gj
    tw_val = jnp.where(validf > 0, jnp.log(gw / aw_sel), 0.0)
    th_val = jnp.where(validf > 0, jnp.log(gh / ah_sel), 0.0)

    cell_i = gj.astype(jnp.int32) * _NW + gi.astype(jnp.int32)
    cell_full = bn * _S + cell_i

    cf = cell_full.astype(f32)
    ones_c = jnp.ones((_NT, 1), f32)
    cell_row = _dot(ones_c, cf, 1, 1)
    valid_row = _dot(ones_c, validf, 1, 1)
    dup = jnp.sum(jnp.where((cell_row == cf) & (valid_row > 0)
                            & (c_i > r_i), 1.0, 0.0), axis=1, keepdims=True)
    winf = validf * (dup == 0).astype(f32)

    as_io = lax.broadcasted_iota(jnp.int32, (_NT, _NAS), 1)
    oh = (as_io == cell_full).astype(f32)

    s_col = lax.broadcasted_iota(jnp.int32, (1, _NAS), 1)
    sp = s_col % _S
    fi = (sp % _NW).astype(f32)
    fj = (sp // _NW).astype(f32)
    awc = jnp.zeros((1, _NAS), f32)
    ahc = jnp.zeros((1, _NAS), f32)
    for a in range(_NA):
        sel = (s_col // _S) == a
        awc = jnp.where(sel, _ANCHORS[2 * a], awc)
        ahc = jnp.where(sel, _ANCHORS[2 * a + 1], ahc)

    def cat(c):
        return jnp.concatenate(
            [dref[0, _CH * a + c:_CH * a + c + 1, :] for a in range(_NA)],
            axis=1)
    xr = cat(0)
    yr = cat(1)
    wr = cat(2)
    hr = cat(3)
    cr = cat(4)
    x = 1.0 / (1.0 + jnp.exp(-xr))
    y = 1.0 / (1.0 + jnp.exp(-yr))
    conf = 1.0 / (1.0 + jnp.exp(-cr))
    px = x + fi
    py = y + fj
    pw = jnp.exp(wr) * awc
    ph = jnp.exp(hr) * ahc

    pxc = _dot(oh, px, 1, 1)
    pyc = _dot(oh, py, 1, 1)
    pwc = _dot(oh, pw, 1, 1)
    phc = _dot(oh, ph, 1, 1)
    iou_val = _iou(gx, gy, gw, gh, pxc, pyc, pwc, phc)
    iou_val = jnp.where(validf > 0, iou_val, 0.0)

    onesf = jnp.ones((_NT, 1), f32)
    V = jnp.concatenate(
        [onesf, tx_val, ty_val, tw_val, th_val, iou_val], axis=1)

    gxz = gx * validf
    gyz = gy * validf
    gwz = gw * validf
    ghz = gh * validf
    plf = px - pw * 0.5
    prt = px + pw * 0.5
    ptp = py - ph * 0.5
    pbt = py + ph * 0.5
    pat = _THRESH * (pw * ph)
    glf = gxz - gwz * 0.5
    grt = gxz + gwz * 0.5
    gtp = gyz - ghz * 0.5
    gbt = gyz + ghz * 0.5
    gat = _THRESH * (gwz * ghz)
    cw = jnp.minimum(prt, grt) - jnp.maximum(plf, glf)
    ch = jnp.minimum(pbt, gbt) - jnp.maximum(ptp, gtp)
    hit = (cw > 0) & (ch > 0) & (cw * ch * (1.0 + _THRESH) > pat + gat)
    noobj = jnp.where(jnp.max(hit.astype(f32), axis=0, keepdims=True) > 0,
                      0.0, 1.0)

    D = _dot(V * winf, oh, 0, 0)
    obj = D[0:1]
    txd = D[1:2] + 0.5 * (1.0 - obj)
    tyd = D[2:3] + 0.5 * (1.0 - obj)
    twd = D[3:4]
    thd = D[4:5]
    tcf = D[5:6]
    cmask = jnp.where(obj > 0, _OBJ_SCALE, noobj)

    lx = jnp.sum((x - txd) ** 2)
    ly = jnp.sum((y - tyd) ** 2)
    lw = jnp.sum((wr - twd) ** 2)
    lh = jnp.sum((hr - thd) ** 2)
    lconf = jnp.sum(cmask * (conf - tcf) ** 2)
    total = 0.5 * (lx + ly + lw + lh + lconf)

    cls_all = jnp.concatenate(
        [dref[0, _CH * a + 5:_CH * a + _CH, :] for a in range(_NA)],
        axis=1)
    L = _dot(oh, cls_all, 1, 1)
    mxL = jnp.max(L, axis=1, keepdims=True)
    seL = jnp.sum(jnp.exp(L - mxL), axis=1, keepdims=True)
    lseL = mxL + jnp.log(seL)
    c50 = lax.broadcasted_iota(jnp.int32, (_NT, _NC), 1)
    pickv = jnp.sum(jnp.where(c50 == tcls.astype(jnp.int32), L, 0.0),
                    axis=1, keepdims=True)
    total = total + jnp.sum(winf * (lseL - pickv))

    @pl.when(b == 0)
    def _init():
        acc[...] = jnp.zeros((1, 1), f32)

    acc[...] += total.reshape(1, 1)


def kernel(output, target):
    out3 = output.reshape(_NB, _NA * _CH, _S)
    tgt3 = target.reshape(_NB, _NT, 5)
    res = pl.pallas_call(
        _body,
        grid=(_NB,),
        in_specs=[
            pl.BlockSpec((1, _NA * _CH, _S), lambda b: (b, 0, 0)),
            pl.BlockSpec((1, _NT, 5), lambda b: (b, 0, 0)),
        ],
        out_specs=pl.BlockSpec((1, 1), lambda b: (0, 0)),
        out_shape=jax.ShapeDtypeStruct((1, 1), jnp.float32),
    )(out3, tgt3)
    return res[0, 0]
